# baseline (speedup 1.0000x reference)
import numpy as np
import jax
import jax.numpy as jnp
from jax import lax
from jax.experimental import pallas as pl
from jax.experimental.pallas import tpu as pltpu
from jax.experimental.pallas import tpu_sc as plsc

N_LEVELS = 16
F = 2
LOG2_T = 19
T = 2 ** LOG2_T
BASE_RES = 16
PER_LEVEL_SCALE = 1.3819129
PRIMES = (1, 2654435761, 805459861)

NW = 32
C = 64
NCOL = 3 + N_LEVELS * F
NR = N_LEVELS * 8


def _levels():
    out = []
    for l in range(N_LEVELS):
        res = int(np.floor(BASE_RES * (PER_LEVEL_SCALE ** l)))
        stride = res + 1
        out.append((res, stride, stride ** 3 <= T, l * T))
    return out


LEVELS = _levels()


def _grid_coords(x, y, z, res):
    rf = jnp.float32(res)
    sx, sy, sz = x * rf, y * rf, z * rf
    ix = sx.astype(jnp.int32)
    iy = sy.astype(jnp.int32)
    iz = sz.astype(jnp.int32)
    return sx, sy, sz, ix, iy, iz


RELAYOUT_CH = 16384


def _relayout_body(tsrc, tdst, src_loc, dst_loc):
    wid = lax.axis_index("s") * 2 + lax.axis_index("c")
    span = tsrc.shape[0] // NW
    base = wid * span
    iota = lax.iota(jnp.int32, 16)
    io2 = iota * 2

    def chunk(c, carry):
        off = base + c * RELAYOUT_CH
        pltpu.sync_copy(tsrc.at[pl.ds(off, RELAYOUT_CH)], src_loc)

        def tile(t, c2):
            tb = t * 256
            for k in range(8):
                f0 = src_loc[pl.ds(tb + k * 16, 16)]
                f1 = src_loc[pl.ds(tb + 128 + k * 16, 16)]
                di = io2 + (tb + k * 32)
                plsc.store_scatter(dst_loc, [di], f0)
                plsc.store_scatter(dst_loc, [di + 1], f1)
            return c2

        lax.fori_loop(0, RELAYOUT_CH // 256, tile, 0)
        pltpu.sync_copy(dst_loc, tdst.at[pl.ds(off, RELAYOUT_CH)])
        return carry

    lax.fori_loop(0, span // RELAYOUT_CH, chunk, 0)


H0 = tuple(enumerate(LEVELS))[:8]
H1 = tuple(enumerate(LEVELS))[8:]


def _body(xyzf, tbl, out, xyz_loc, idx_buf, fv_buf, w_buf, rows, obuf,
          sems, sem_x):
    wid = lax.axis_index("s") * 2 + lax.axis_index("c")
    npts = xyzf.shape[0] // 3
    per_w = npts // NW
    nblk = per_w // C
    iota = lax.iota(jnp.int32, 16)
    base0 = wid * per_w

    def phase_a(levels, pb):
        pbv = jnp.zeros((16,), jnp.int32) + pb
        for l, (res, stride, dense, lbase) in levels:

            def grp_a(g, c2, res=res, stride=stride, dense=dense,
                      lbase=lbase, l=l):
                for so in (0, 16):
                    o = g * 32 + so
                    p3 = (iota + o) * 3
                    x = plsc.load_gather(xyz_loc, [pbv, p3])
                    y = plsc.load_gather(xyz_loc, [pbv, p3 + 1])
                    z = plsc.load_gather(xyz_loc, [pbv, p3 + 2])
                    sx, sy, sz, ix, iy, iz = _grid_coords(x, y, z, res)
                    w_buf[l * 3, pl.ds(o, 16)] = sx - ix.astype(jnp.float32)
                    w_buf[l * 3 + 1, pl.ds(o, 16)] = sy - iy.astype(jnp.float32)
                    w_buf[l * 3 + 2, pl.ds(o, 16)] = sz - iz.astype(jnp.float32)
                    if dense:
                        s2 = stride * stride
                        b000 = ix + iy * stride + iz * s2 + lbase
                        for corner in range(8):
                            off = ((corner & 1) + ((corner >> 1) & 1) * stride
                                   + ((corner >> 2) & 1) * s2)
                            e = b000 + off
                            r4, oc = divmod(l * 8 + corner, 4)
                            idx_buf[r4, pl.ds(oc * C + o, 16)] = e >> 2
                            fv_buf[r4, pl.ds(oc * C + o, 16)] = (e & 3) << 1
                    else:
                        ux = ix.astype(jnp.uint32)
                        uy = iy.astype(jnp.uint32)
                        uz = iz.astype(jnp.uint32)
                        p1 = jnp.uint32(PRIMES[1])
                        p2 = jnp.uint32(PRIMES[2])
                        hy0 = uy * p1
                        hy1 = hy0 + p1
                        hz0 = uz * p2
                        hz1 = hz0 + p2
                        hx1 = ux + jnp.uint32(1)
                        mask = jnp.uint32(T - 1)
                        for corner in range(8):
                            hx = hx1 if (corner & 1) else ux
                            hy = hy1 if (corner & 2) else hy0
                            hz = hz1 if (corner & 4) else hz0
                            h = (hx ^ hy ^ hz) & mask
                            e = h.astype(jnp.int32) + lbase
                            r4, oc = divmod(l * 8 + corner, 4)
                            idx_buf[r4, pl.ds(oc * C + o, 16)] = e >> 2
                            fv_buf[r4, pl.ds(oc * C + o, 16)] = (e & 3) << 1
                return c2

            lax.fori_loop(0, C // 32, grp_a, 0)
            for j in range(2):
                r4 = l * 2 + j
                pltpu.async_copy(tbl.at[idx_buf.at[r4]], rows.at[r4],
                                 sems.at[l])

    def drain_half(levels):
        for l, _ in levels:
            for j in range(2):
                r4 = l * 2 + j
                pltpu.make_async_copy(
                    tbl.at[idx_buf.at[r4]], rows.at[r4], sems.at[l]).wait()

    def phase_c(levels):
        for l, _ in levels:

            def grp_c(g, c2, l=l):
                for so in (0, 16):
                    o = g * 32 + so
                    pv = iota + o
                    fx = w_buf[l * 3, pl.ds(o, 16)]
                    fy = w_buf[l * 3 + 1, pl.ds(o, 16)]
                    fz = w_buf[l * 3 + 2, pl.ds(o, 16)]
                    gx, gy, gz = 1.0 - fx, 1.0 - fy, 1.0 - fz
                    wxy = (gx * gy, fx * gy, gx * fy, fx * fy)
                    acc0 = acc1 = None
                    for corner in range(8):
                        wc = wxy[corner & 3] * (fz if (corner & 4) else gz)
                        r4, oc = divmod(l * 8 + corner, 4)
                        rv = jnp.full((16,), r4, jnp.int32)
                        pv4 = pv + oc * C
                        fv = fv_buf[r4, pl.ds(oc * C + o, 16)]
                        f0 = plsc.load_gather(rows, [rv, pv4, fv])
                        f1 = plsc.load_gather(rows, [rv, pv4, fv + 1])
                        if corner == 0:
                            acc0, acc1 = f0 * wc, f1 * wc
                        else:
                            acc0, acc1 = acc0 + f0 * wc, acc1 + f1 * wc
                    obuf[3 + 2 * l, pl.ds(o, 16)] = acc0
                    obuf[4 + 2 * l, pl.ds(o, 16)] = acc1
                return c2

            lax.fori_loop(0, C // 32, grp_c, 0)

    def finish_xyz(pb):
        pbv = jnp.zeros((16,), jnp.int32) + pb

        def grp_x(g, c2):
            o = g * 16
            pv = iota + o
            p3 = pv * 3
            x = plsc.load_gather(xyz_loc, [pbv, p3])
            y = plsc.load_gather(xyz_loc, [pbv, p3 + 1])
            z = plsc.load_gather(xyz_loc, [pbv, p3 + 2])
            obuf[0, pl.ds(o, 16)] = x * 2.0 - 1.0
            obuf[1, pl.ds(o, 16)] = y * 2.0 - 1.0
            obuf[2, pl.ds(o, 16)] = z * 2.0 - 1.0
            return c2

        lax.fori_loop(0, C // 16, grp_x, 0)

    pltpu.sync_copy(xyzf.at[pl.ds(base0 * 3, C * 3)], xyz_loc.at[0])
    phase_a(H0, 0)

    def block(b, carry):
        pb = b & 1
        base = base0 + b * C
        bn = jnp.minimum(b + 1, nblk - 1)
        basen = base0 + bn * C

        phase_a(H1, pb)
        xcp = pltpu.async_copy(
            xyzf.at[pl.ds(basen * 3, C * 3)], xyz_loc.at[1 - pb], sem_x)
        drain_half(H0)
        phase_c(H0)
        finish_xyz(pb)
        pltpu.make_async_copy(
            xyzf.at[pl.ds(basen * 3, C * 3)], xyz_loc.at[1 - pb], sem_x).wait()
        phase_a(H0, 1 - pb)
        drain_half(H1)
        phase_c(H1)
        pltpu.sync_copy(obuf, out.at[:, pl.ds(base, C)])
        return carry

    lax.fori_loop(0, nblk, block, 0)

    drain_half(H0)


def kernel(xyz, table):
    n = xyz.shape[0]
    nw = N_LEVELS * T * F
    tnative = (table.reshape(N_LEVELS, T // 128, 128, F)
               .transpose(0, 1, 3, 2).reshape(nw))
    xyzf = xyz.reshape(n * 3)
    mesh = plsc.VectorSubcoreMesh(core_axis_name="c", subcore_axis_name="s")
    k1 = pl.kernel(
        _relayout_body,
        out_type=jax.ShapeDtypeStruct((nw,), jnp.float32),
        mesh=mesh,
        scratch_types=[
            pltpu.VMEM((RELAYOUT_CH,), jnp.float32),
            pltpu.VMEM((RELAYOUT_CH,), jnp.float32),
        ],
        compiler_params=pltpu.CompilerParams(
            needs_layout_passes=False, use_tc_tiling_on_sc=False),
    )
    tbl = k1(tnative).reshape(nw // 8, 8)
    k = pl.kernel(
        _body,
        out_type=jax.ShapeDtypeStruct((NCOL, n), jnp.float32),
        mesh=mesh,
        scratch_types=[
            pltpu.VMEM((2, C * 3), jnp.float32),
            pltpu.VMEM((NR // 4, C * 4), jnp.int32),
            pltpu.VMEM((NR // 4, C * 4), jnp.int32),
            pltpu.VMEM((N_LEVELS * 3, C), jnp.float32),
            pltpu.VMEM((NR // 4, C * 4, 8), jnp.float32),
            pltpu.VMEM((NCOL, C), jnp.float32),
            pltpu.SemaphoreType.DMA((N_LEVELS,)),
            pltpu.SemaphoreType.DMA,
        ],
        compiler_params=pltpu.CompilerParams(
            needs_layout_passes=False, use_tc_tiling_on_sc=False),
    )
    return k(xyzf, tbl).T


# speedup vs baseline: 1.3465x; 1.3465x over previous
import numpy as np
import jax
import jax.numpy as jnp
from jax import lax
from jax.experimental import pallas as pl
from jax.experimental.pallas import tpu as pltpu
from jax.experimental.pallas import tpu_sc as plsc

N_LEVELS = 16
F = 2
LOG2_T = 19
T = 2 ** LOG2_T
BASE_RES = 16
PER_LEVEL_SCALE = 1.3819129
PRIMES = (1, 2654435761, 805459861)

NW = 32
C = 64
NCOL = 3 + N_LEVELS * F
NR = N_LEVELS * 8


def _levels():
    out = []
    for l in range(N_LEVELS):
        res = int(np.floor(BASE_RES * (PER_LEVEL_SCALE ** l)))
        stride = res + 1
        out.append((res, stride, stride ** 3 <= T, l * T))
    return out


LEVELS = _levels()


def _grid_coords(x, y, z, res):
    rf = jnp.float32(res)
    sx, sy, sz = x * rf, y * rf, z * rf
    ix = sx.astype(jnp.int32)
    iy = sy.astype(jnp.int32)
    iz = sz.astype(jnp.int32)
    return sx, sy, sz, ix, iy, iz


RELAYOUT_CH = 16384


def _relayout_body(tsrc, tdst, src_loc, dst_loc):
    wid = lax.axis_index("s") * 2 + lax.axis_index("c")
    span = tsrc.shape[0] // NW
    base = wid * span
    iota = lax.iota(jnp.int32, 16)
    io2 = iota * 2

    def chunk(c, carry):
        off = base + c * RELAYOUT_CH
        pltpu.sync_copy(tsrc.at[pl.ds(off, RELAYOUT_CH)], src_loc)

        def tile(t, c2):
            tb = t * 256
            for k in range(8):
                f0 = src_loc[pl.ds(tb + k * 16, 16)]
                f1 = src_loc[pl.ds(tb + 128 + k * 16, 16)]
                di = io2 + (tb + k * 32)
                plsc.store_scatter(dst_loc, [di], f0)
                plsc.store_scatter(dst_loc, [di + 1], f1)
            return c2

        lax.fori_loop(0, RELAYOUT_CH // 256, tile, 0)
        pltpu.sync_copy(dst_loc, tdst.at[pl.ds(off, RELAYOUT_CH)])
        return carry

    lax.fori_loop(0, span // RELAYOUT_CH, chunk, 0)


H0 = tuple(enumerate(LEVELS))[:8]
H1 = tuple(enumerate(LEVELS))[8:]


def _body(xyzf, tbl, out, xyz_loc, idx_buf, fv_buf, w_buf, rows, obuf,
          sems, sem_x):
    wid = lax.axis_index("s") * 2 + lax.axis_index("c")
    npts = xyzf.shape[0] // 3
    per_w = npts // NW
    nblk = per_w // C
    iota = lax.iota(jnp.int32, 16)
    base0 = wid * per_w

    def phase_a(levels, pb):
        pbv = jnp.zeros((16,), jnp.int32) + pb
        for l, (res, stride, dense, lbase) in levels:

            def grp_a(g, c2, res=res, stride=stride, dense=dense,
                      lbase=lbase, l=l):
                for so in (0, 16):
                    o = g * 32 + so
                    p3 = (iota + o) * 3
                    x = plsc.load_gather(xyz_loc, [pbv, p3])
                    y = plsc.load_gather(xyz_loc, [pbv, p3 + 1])
                    z = plsc.load_gather(xyz_loc, [pbv, p3 + 2])
                    sx, sy, sz, ix, iy, iz = _grid_coords(x, y, z, res)
                    w_buf[l * 3, pl.ds(o, 16)] = sx - ix.astype(jnp.float32)
                    w_buf[l * 3 + 1, pl.ds(o, 16)] = sy - iy.astype(jnp.float32)
                    w_buf[l * 3 + 2, pl.ds(o, 16)] = sz - iz.astype(jnp.float32)
                    if dense:
                        s2 = stride * stride
                        b000 = ix + iy * stride + iz * s2 + lbase
                        for corner in range(8):
                            off = ((corner & 1) + ((corner >> 1) & 1) * stride
                                   + ((corner >> 2) & 1) * s2)
                            e = b000 + off
                            r4, oc = divmod(l * 8 + corner, 4)
                            idx_buf[r4, pl.ds(oc * C + o, 16)] = e >> 2
                            fv_buf[r4, pl.ds(oc * C + o, 16)] = (e & 3) << 1
                    else:
                        ux = ix.astype(jnp.uint32)
                        uy = iy.astype(jnp.uint32)
                        uz = iz.astype(jnp.uint32)
                        p1 = jnp.uint32(PRIMES[1])
                        p2 = jnp.uint32(PRIMES[2])
                        hy0 = uy * p1
                        hy1 = hy0 + p1
                        hz0 = uz * p2
                        hz1 = hz0 + p2
                        hx1 = ux + jnp.uint32(1)
                        mask = jnp.uint32(T - 1)
                        for corner in range(8):
                            hx = hx1 if (corner & 1) else ux
                            hy = hy1 if (corner & 2) else hy0
                            hz = hz1 if (corner & 4) else hz0
                            h = (hx ^ hy ^ hz) & mask
                            e = h.astype(jnp.int32) + lbase
                            r4, oc = divmod(l * 8 + corner, 4)
                            idx_buf[r4, pl.ds(oc * C + o, 16)] = e >> 2
                            fv_buf[r4, pl.ds(oc * C + o, 16)] = (e & 3) << 1
                return c2

            lax.fori_loop(0, C // 32, grp_a, 0)
            for j in range(2):
                r4 = l * 2 + j
                pltpu.async_copy(tbl.at[idx_buf.at[r4]], rows.at[r4],
                                 sems.at[l])

    def drain_half(levels):
        for l, _ in levels:
            for j in range(2):
                r4 = l * 2 + j
                pltpu.make_async_copy(
                    tbl.at[idx_buf.at[r4]], rows.at[r4], sems.at[l]).wait()

    def phase_c(levels):
        for l, _ in levels:

            def grp_c(g, c2, l=l):
                for so in (0, 16):
                    o = g * 32 + so
                    pv = iota + o
                    pcol = pv * NCOL
                    fx = w_buf[l * 3, pl.ds(o, 16)]
                    fy = w_buf[l * 3 + 1, pl.ds(o, 16)]
                    fz = w_buf[l * 3 + 2, pl.ds(o, 16)]
                    gx, gy, gz = 1.0 - fx, 1.0 - fy, 1.0 - fz
                    wxy = (gx * gy, fx * gy, gx * fy, fx * fy)
                    acc0 = acc1 = None
                    for corner in range(8):
                        wc = wxy[corner & 3] * (fz if (corner & 4) else gz)
                        r4, oc = divmod(l * 8 + corner, 4)
                        rv = jnp.full((16,), r4, jnp.int32)
                        pv4 = pv + oc * C
                        fv = fv_buf[r4, pl.ds(oc * C + o, 16)]
                        f0 = plsc.load_gather(rows, [rv, pv4, fv])
                        f1 = plsc.load_gather(rows, [rv, pv4, fv + 1])
                        if corner == 0:
                            acc0, acc1 = f0 * wc, f1 * wc
                        else:
                            acc0, acc1 = acc0 + f0 * wc, acc1 + f1 * wc
                    plsc.store_scatter(obuf, [pcol + (3 + 2 * l)], acc0)
                    plsc.store_scatter(obuf, [pcol + (4 + 2 * l)], acc1)
                return c2

            lax.fori_loop(0, C // 32, grp_c, 0)

    def finish_xyz(pb):
        pbv = jnp.zeros((16,), jnp.int32) + pb

        def grp_x(g, c2):
            o = g * 16
            pv = iota + o
            p3 = pv * 3
            pcol = pv * NCOL
            x = plsc.load_gather(xyz_loc, [pbv, p3])
            y = plsc.load_gather(xyz_loc, [pbv, p3 + 1])
            z = plsc.load_gather(xyz_loc, [pbv, p3 + 2])
            plsc.store_scatter(obuf, [pcol], x * 2.0 - 1.0)
            plsc.store_scatter(obuf, [pcol + 1], y * 2.0 - 1.0)
            plsc.store_scatter(obuf, [pcol + 2], z * 2.0 - 1.0)
            return c2

        lax.fori_loop(0, C // 16, grp_x, 0)

    pltpu.sync_copy(xyzf.at[pl.ds(base0 * 3, C * 3)], xyz_loc.at[0])
    phase_a(H0, 0)

    def block(b, carry):
        pb = b & 1
        base = base0 + b * C
        bn = jnp.minimum(b + 1, nblk - 1)
        basen = base0 + bn * C

        phase_a(H1, pb)
        xcp = pltpu.async_copy(
            xyzf.at[pl.ds(basen * 3, C * 3)], xyz_loc.at[1 - pb], sem_x)
        drain_half(H0)
        phase_c(H0)
        finish_xyz(pb)
        pltpu.make_async_copy(
            xyzf.at[pl.ds(basen * 3, C * 3)], xyz_loc.at[1 - pb], sem_x).wait()
        phase_a(H0, 1 - pb)
        drain_half(H1)
        phase_c(H1)
        pltpu.sync_copy(obuf, out.at[pl.ds(base * NCOL, C * NCOL)])
        return carry

    lax.fori_loop(0, nblk, block, 0)

    drain_half(H0)


def kernel(xyz, table):
    n = xyz.shape[0]
    nw = N_LEVELS * T * F
    tnative = (table.reshape(N_LEVELS, T // 128, 128, F)
               .transpose(0, 1, 3, 2).reshape(nw))
    xyzf = xyz.reshape(n * 3)
    mesh = plsc.VectorSubcoreMesh(core_axis_name="c", subcore_axis_name="s")
    k1 = pl.kernel(
        _relayout_body,
        out_type=jax.ShapeDtypeStruct((nw,), jnp.float32),
        mesh=mesh,
        scratch_types=[
            pltpu.VMEM((RELAYOUT_CH,), jnp.float32),
            pltpu.VMEM((RELAYOUT_CH,), jnp.float32),
        ],
        compiler_params=pltpu.CompilerParams(
            needs_layout_passes=False, use_tc_tiling_on_sc=False),
    )
    tbl = k1(tnative).reshape(nw // 8, 8)
    k = pl.kernel(
        _body,
        out_type=jax.ShapeDtypeStruct((n * NCOL,), jnp.float32),
        mesh=mesh,
        scratch_types=[
            pltpu.VMEM((2, C * 3), jnp.float32),
            pltpu.VMEM((NR // 4, C * 4), jnp.int32),
            pltpu.VMEM((NR // 4, C * 4), jnp.int32),
            pltpu.VMEM((N_LEVELS * 3, C), jnp.float32),
            pltpu.VMEM((NR // 4, C * 4, 8), jnp.float32),
            pltpu.VMEM((C * NCOL,), jnp.float32),
            pltpu.SemaphoreType.DMA((N_LEVELS,)),
            pltpu.SemaphoreType.DMA,
        ],
        compiler_params=pltpu.CompilerParams(
            needs_layout_passes=False, use_tc_tiling_on_sc=False),
    )
    return k(xyzf, tbl).reshape(n, NCOL)


# speedup vs baseline: 1.5040x; 1.1169x over previous
import numpy as np
import jax
import jax.numpy as jnp
from jax import lax
from jax.experimental import pallas as pl
from jax.experimental.pallas import tpu as pltpu
from jax.experimental.pallas import tpu_sc as plsc

N_LEVELS = 16
F = 2
LOG2_T = 19
T = 2 ** LOG2_T
BASE_RES = 16
PER_LEVEL_SCALE = 1.3819129
PRIMES = (1, 2654435761, 805459861)

NW = 32
C = 64
NCOL = 3 + N_LEVELS * F
NR = N_LEVELS * 8


def _levels():
    out = []
    for l in range(N_LEVELS):
        res = int(np.floor(BASE_RES * (PER_LEVEL_SCALE ** l)))
        stride = res + 1
        out.append((res, stride, stride ** 3 <= T, l * T))
    return out


LEVELS = _levels()


def _grid_coords(x, y, z, res):
    rf = jnp.float32(res)
    sx, sy, sz = x * rf, y * rf, z * rf
    ix = sx.astype(jnp.int32)
    iy = sy.astype(jnp.int32)
    iz = sz.astype(jnp.int32)
    return sx, sy, sz, ix, iy, iz


RELAYOUT_CH = 16384


def _relayout_body(tsrc, tdst, src_loc, dst_loc):
    wid = lax.axis_index("s") * 2 + lax.axis_index("c")
    span = tsrc.shape[0] // NW
    base = wid * span
    iota = lax.iota(jnp.int32, 16)
    io2 = iota * 2

    def chunk(c, carry):
        off = base + c * RELAYOUT_CH
        pltpu.sync_copy(tsrc.at[pl.ds(off, RELAYOUT_CH)], src_loc)

        def tile(t, c2):
            tb = t * 256
            for k in range(8):
                f0 = src_loc[pl.ds(tb + k * 16, 16)]
                f1 = src_loc[pl.ds(tb + 128 + k * 16, 16)]
                di = io2 + (tb + k * 32)
                plsc.store_scatter(dst_loc, [di], f0)
                plsc.store_scatter(dst_loc, [di + 1], f1)
            return c2

        lax.fori_loop(0, RELAYOUT_CH // 256, tile, 0)
        pltpu.sync_copy(dst_loc, tdst.at[pl.ds(off, RELAYOUT_CH)])
        return carry

    lax.fori_loop(0, span // RELAYOUT_CH, chunk, 0)


H0 = tuple(enumerate(LEVELS))[:8]
H1 = tuple(enumerate(LEVELS))[8:]

N_RES = 2
TLOC_BASE = (0, 9832)
TLOC_SIZE = 34168
RES_ADJ = tuple(TLOC_BASE[l] - l * T * F for l in range(N_RES))


def _body(xyzf, tbl, out, xyz_loc, idx_buf, fv_buf, w_buf, rows, obuf,
          tloc, sems, sem_x):
    wid = lax.axis_index("s") * 2 + lax.axis_index("c")
    npts = xyzf.shape[0] // 3
    per_w = npts // NW
    nblk = per_w // C
    iota = lax.iota(jnp.int32, 16)
    base0 = wid * per_w

    def phase_a(levels, pb):
        pbv = jnp.zeros((16,), jnp.int32) + pb
        for l, (res, stride, dense, lbase) in levels:

            def grp_a(g, c2, res=res, stride=stride, dense=dense,
                      lbase=lbase, l=l):
                for so in (0, 16):
                    o = g * 32 + so
                    p3 = (iota + o) * 3
                    x = plsc.load_gather(xyz_loc, [pbv, p3])
                    y = plsc.load_gather(xyz_loc, [pbv, p3 + 1])
                    z = plsc.load_gather(xyz_loc, [pbv, p3 + 2])
                    sx, sy, sz, ix, iy, iz = _grid_coords(x, y, z, res)
                    w_buf[l * 3, pl.ds(o, 16)] = sx - ix.astype(jnp.float32)
                    w_buf[l * 3 + 1, pl.ds(o, 16)] = sy - iy.astype(jnp.float32)
                    w_buf[l * 3 + 2, pl.ds(o, 16)] = sz - iz.astype(jnp.float32)
                    if dense:
                        s2 = stride * stride
                        b000 = ix + iy * stride + iz * s2 + lbase
                        for corner in range(8):
                            off = ((corner & 1) + ((corner >> 1) & 1) * stride
                                   + ((corner >> 2) & 1) * s2)
                            e = b000 + off
                            r4, oc = divmod(l * 8 + corner, 4)
                            if l < N_RES:
                                ev = e * 2 + RES_ADJ[l]
                                idx_buf[r4, pl.ds(oc * C + o, 16)] = ev >> 3
                                fv_buf[r4, pl.ds(oc * C + o, 16)] = ev & 7
                            else:
                                idx_buf[r4, pl.ds(oc * C + o, 16)] = e >> 2
                                fv_buf[r4, pl.ds(oc * C + o, 16)] = (e & 3) << 1
                    else:
                        ux = ix.astype(jnp.uint32)
                        uy = iy.astype(jnp.uint32)
                        uz = iz.astype(jnp.uint32)
                        p1 = jnp.uint32(PRIMES[1])
                        p2 = jnp.uint32(PRIMES[2])
                        hy0 = uy * p1
                        hy1 = hy0 + p1
                        hz0 = uz * p2
                        hz1 = hz0 + p2
                        hx1 = ux + jnp.uint32(1)
                        mask = jnp.uint32(T - 1)
                        for corner in range(8):
                            hx = hx1 if (corner & 1) else ux
                            hy = hy1 if (corner & 2) else hy0
                            hz = hz1 if (corner & 4) else hz0
                            h = (hx ^ hy ^ hz) & mask
                            e = h.astype(jnp.int32) + lbase
                            r4, oc = divmod(l * 8 + corner, 4)
                            idx_buf[r4, pl.ds(oc * C + o, 16)] = e >> 2
                            fv_buf[r4, pl.ds(oc * C + o, 16)] = (e & 3) << 1
                return c2

            lax.fori_loop(0, C // 32, grp_a, 0)
            if l >= N_RES:
                for j in range(2):
                    r4 = l * 2 + j
                    pltpu.async_copy(tbl.at[idx_buf.at[r4]], rows.at[r4],
                                     sems.at[l])

    def drain_half(levels):
        for l, _ in levels:
            if l < N_RES:
                continue
            for j in range(2):
                r4 = l * 2 + j
                pltpu.make_async_copy(
                    tbl.at[idx_buf.at[r4]], rows.at[r4], sems.at[l]).wait()

    def phase_c(levels):
        for l, _ in levels:

            def grp_c(g, c2, l=l):
                for so in (0, 16):
                    o = g * 32 + so
                    pv = iota + o
                    pcol = pv * NCOL
                    fx = w_buf[l * 3, pl.ds(o, 16)]
                    fy = w_buf[l * 3 + 1, pl.ds(o, 16)]
                    fz = w_buf[l * 3 + 2, pl.ds(o, 16)]
                    gx, gy, gz = 1.0 - fx, 1.0 - fy, 1.0 - fz
                    wxy = (gx * gy, fx * gy, gx * fy, fx * fy)
                    acc0 = acc1 = None
                    for corner in range(8):
                        wc = wxy[corner & 3] * (fz if (corner & 4) else gz)
                        r4, oc = divmod(l * 8 + corner, 4)
                        if l < N_RES:
                            er = idx_buf[r4, pl.ds(oc * C + o, 16)]
                            ew = fv_buf[r4, pl.ds(oc * C + o, 16)]
                            f0 = plsc.load_gather(tloc, [er, ew])
                            f1 = plsc.load_gather(tloc, [er, ew + 1])
                        else:
                            rv = jnp.full((16,), r4, jnp.int32)
                            pv4 = pv + oc * C
                            fv = fv_buf[r4, pl.ds(oc * C + o, 16)]
                            f0 = plsc.load_gather(rows, [rv, pv4, fv])
                            f1 = plsc.load_gather(rows, [rv, pv4, fv + 1])
                        if corner == 0:
                            acc0, acc1 = f0 * wc, f1 * wc
                        else:
                            acc0, acc1 = acc0 + f0 * wc, acc1 + f1 * wc
                    plsc.store_scatter(obuf, [pcol + (3 + 2 * l)], acc0)
                    plsc.store_scatter(obuf, [pcol + (4 + 2 * l)], acc1)
                return c2

            lax.fori_loop(0, C // 32, grp_c, 0)

    def finish_xyz(pb):
        pbv = jnp.zeros((16,), jnp.int32) + pb

        def grp_x(g, c2):
            o = g * 16
            pv = iota + o
            p3 = pv * 3
            pcol = pv * NCOL
            x = plsc.load_gather(xyz_loc, [pbv, p3])
            y = plsc.load_gather(xyz_loc, [pbv, p3 + 1])
            z = plsc.load_gather(xyz_loc, [pbv, p3 + 2])
            plsc.store_scatter(obuf, [pcol], x * 2.0 - 1.0)
            plsc.store_scatter(obuf, [pcol + 1], y * 2.0 - 1.0)
            plsc.store_scatter(obuf, [pcol + 2], z * 2.0 - 1.0)
            return c2

        lax.fori_loop(0, C // 16, grp_x, 0)

    for l in range(N_RES):
        nrows = ((LEVELS[l][1] ** 3) * F + 7) // 8
        pltpu.sync_copy(tbl.at[pl.ds(l * T * F // 8, nrows)],
                        tloc.at[pl.ds(TLOC_BASE[l] // 8, nrows)])
    pltpu.sync_copy(xyzf.at[pl.ds(base0 * 3, C * 3)], xyz_loc.at[0])
    phase_a(H0, 0)

    def block(b, carry):
        pb = b & 1
        base = base0 + b * C
        bn = jnp.minimum(b + 1, nblk - 1)
        basen = base0 + bn * C

        phase_a(H1, pb)
        xcp = pltpu.async_copy(
            xyzf.at[pl.ds(basen * 3, C * 3)], xyz_loc.at[1 - pb], sem_x)
        drain_half(H0)
        phase_c(H0)
        finish_xyz(pb)
        pltpu.make_async_copy(
            xyzf.at[pl.ds(basen * 3, C * 3)], xyz_loc.at[1 - pb], sem_x).wait()
        phase_a(H0, 1 - pb)
        drain_half(H1)
        phase_c(H1)
        pltpu.sync_copy(obuf, out.at[pl.ds(base * NCOL, C * NCOL)])
        return carry

    lax.fori_loop(0, nblk, block, 0)

    drain_half(H0)


def kernel(xyz, table):
    n = xyz.shape[0]
    nw = N_LEVELS * T * F
    tnative = (table.reshape(N_LEVELS, T // 128, 128, F)
               .transpose(0, 1, 3, 2).reshape(nw))
    xyzf = xyz.reshape(n * 3)
    mesh = plsc.VectorSubcoreMesh(core_axis_name="c", subcore_axis_name="s")
    k1 = pl.kernel(
        _relayout_body,
        out_type=jax.ShapeDtypeStruct((nw,), jnp.float32),
        mesh=mesh,
        scratch_types=[
            pltpu.VMEM((RELAYOUT_CH,), jnp.float32),
            pltpu.VMEM((RELAYOUT_CH,), jnp.float32),
        ],
        compiler_params=pltpu.CompilerParams(
            needs_layout_passes=False, use_tc_tiling_on_sc=False),
    )
    tbl = k1(tnative).reshape(nw // 8, 8)
    k = pl.kernel(
        _body,
        out_type=jax.ShapeDtypeStruct((n * NCOL,), jnp.float32),
        mesh=mesh,
        scratch_types=[
            pltpu.VMEM((2, C * 3), jnp.float32),
            pltpu.VMEM((NR // 4, C * 4), jnp.int32),
            pltpu.VMEM((NR // 4, C * 4), jnp.int32),
            pltpu.VMEM((N_LEVELS * 3, C), jnp.float32),
            pltpu.VMEM((NR // 4, C * 4, 8), jnp.float32),
            pltpu.VMEM((C * NCOL,), jnp.float32),
            pltpu.VMEM((TLOC_SIZE // 8, 8), jnp.float32),
            pltpu.SemaphoreType.DMA((N_LEVELS,)),
            pltpu.SemaphoreType.DMA,
        ],
        compiler_params=pltpu.CompilerParams(
            needs_layout_passes=False, use_tc_tiling_on_sc=False),
    )
    return k(xyzf, tbl).reshape(n, NCOL)


# speedup vs baseline: 1.5058x; 1.0012x over previous
import numpy as np
import jax
import jax.numpy as jnp
from jax import lax
from jax.experimental import pallas as pl
from jax.experimental.pallas import tpu as pltpu
from jax.experimental.pallas import tpu_sc as plsc

N_LEVELS = 16
F = 2
LOG2_T = 19
T = 2 ** LOG2_T
BASE_RES = 16
PER_LEVEL_SCALE = 1.3819129
PRIMES = (1, 2654435761, 805459861)

NW = 32
C = 64
NCOL = 3 + N_LEVELS * F
NR = N_LEVELS * 8


def _levels():
    out = []
    for l in range(N_LEVELS):
        res = int(np.floor(BASE_RES * (PER_LEVEL_SCALE ** l)))
        stride = res + 1
        out.append((res, stride, stride ** 3 <= T, l * T))
    return out


LEVELS = _levels()


def _grid_coords(x, y, z, res):
    rf = jnp.float32(res)
    sx, sy, sz = x * rf, y * rf, z * rf
    ix = sx.astype(jnp.int32)
    iy = sy.astype(jnp.int32)
    iz = sz.astype(jnp.int32)
    return sx, sy, sz, ix, iy, iz


RELAYOUT_CH = 16384


def _relayout_body(tsrc, tdst, src_loc, dst_loc):
    wid = lax.axis_index("s") * 2 + lax.axis_index("c")
    span = tsrc.shape[0] // NW
    base = wid * span
    iota = lax.iota(jnp.int32, 16)
    io2 = iota * 2

    def chunk(c, carry):
        off = base + c * RELAYOUT_CH
        pltpu.sync_copy(tsrc.at[pl.ds(off, RELAYOUT_CH)], src_loc)

        def tile(t, c2):
            tb = t * 256
            for k in range(8):
                f0 = src_loc[pl.ds(tb + k * 16, 16)]
                f1 = src_loc[pl.ds(tb + 128 + k * 16, 16)]
                di = io2 + (tb + k * 32)
                plsc.store_scatter(dst_loc, [di], f0)
                plsc.store_scatter(dst_loc, [di + 1], f1)
            return c2

        lax.fori_loop(0, RELAYOUT_CH // 256, tile, 0)
        pltpu.sync_copy(dst_loc, tdst.at[pl.ds(off, RELAYOUT_CH)])
        return carry

    lax.fori_loop(0, span // RELAYOUT_CH, chunk, 0)


H0 = tuple(enumerate(LEVELS))[:8]
H1 = tuple(enumerate(LEVELS))[8:]

N_RES = 2
TLOC_BASE = (0, 9832)
TLOC_SIZE = 34168
RES_ADJ = tuple(TLOC_BASE[l] - l * T * F for l in range(N_RES))


def _body(xyzf, tbl, out, xyz_loc, idx_buf, fv_buf, w_buf, rows, obuf,
          tloc, sems, sem_x):
    wid = lax.axis_index("s") * 2 + lax.axis_index("c")
    npts = xyzf.shape[0] // 3
    per_w = npts // NW
    nblk = per_w // C
    iota = lax.iota(jnp.int32, 16)
    base0 = wid * per_w

    def phase_a(levels, pb):
        pbv = jnp.zeros((16,), jnp.int32) + pb
        for l, (res, stride, dense, lbase) in levels:

            def grp_a(g, c2, res=res, stride=stride, dense=dense,
                      lbase=lbase, l=l):
                for so in (0, 16):
                    o = g * 32 + so
                    p3 = (iota + o) * 3
                    x = plsc.load_gather(xyz_loc, [pbv, p3])
                    y = plsc.load_gather(xyz_loc, [pbv, p3 + 1])
                    z = plsc.load_gather(xyz_loc, [pbv, p3 + 2])
                    sx, sy, sz, ix, iy, iz = _grid_coords(x, y, z, res)
                    w_buf[l * 3, pl.ds(o, 16)] = sx - ix.astype(jnp.float32)
                    w_buf[l * 3 + 1, pl.ds(o, 16)] = sy - iy.astype(jnp.float32)
                    w_buf[l * 3 + 2, pl.ds(o, 16)] = sz - iz.astype(jnp.float32)
                    if dense:
                        s2 = stride * stride
                        b000 = ix + iy * stride + iz * s2 + lbase
                        for corner in range(8):
                            off = ((corner & 1) + ((corner >> 1) & 1) * stride
                                   + ((corner >> 2) & 1) * s2)
                            e = b000 + off
                            r4, oc = divmod(l * 8 + corner, 4)
                            if l < N_RES:
                                ev = e * 2 + RES_ADJ[l]
                                idx_buf[r4, pl.ds(oc * C + o, 16)] = ev >> 3
                                fv_buf[r4, pl.ds(oc * C + o, 16)] = ev & 7
                            else:
                                idx_buf[r4, pl.ds(oc * C + o, 16)] = e >> 2
                                fv_buf[r4, pl.ds(oc * C + o, 16)] = (e & 3) << 1
                    else:
                        ux = ix.astype(jnp.uint32)
                        uy = iy.astype(jnp.uint32)
                        uz = iz.astype(jnp.uint32)
                        p1 = jnp.uint32(PRIMES[1])
                        p2 = jnp.uint32(PRIMES[2])
                        hy0 = uy * p1
                        hy1 = hy0 + p1
                        hz0 = uz * p2
                        hz1 = hz0 + p2
                        hx1 = ux + jnp.uint32(1)
                        mask = jnp.uint32(T - 1)
                        for corner in range(8):
                            hx = hx1 if (corner & 1) else ux
                            hy = hy1 if (corner & 2) else hy0
                            hz = hz1 if (corner & 4) else hz0
                            h = (hx ^ hy ^ hz) & mask
                            e = h.astype(jnp.int32) + lbase
                            r4, oc = divmod(l * 8 + corner, 4)
                            idx_buf[r4, pl.ds(oc * C + o, 16)] = e >> 2
                            fv_buf[r4, pl.ds(oc * C + o, 16)] = (e & 3) << 1
                return c2

            lax.fori_loop(0, C // 32, grp_a, 0)
            if l >= N_RES:
                for j in range(2):
                    r4 = l * 2 + j
                    pltpu.async_copy(tbl.at[idx_buf.at[r4]], rows.at[r4],
                                     sems.at[l])

    def drain_half(levels):
        for l, _ in levels:
            if l < N_RES:
                continue
            for j in range(2):
                r4 = l * 2 + j
                pltpu.make_async_copy(
                    tbl.at[idx_buf.at[r4]], rows.at[r4], sems.at[l]).wait()

    def phase_c(levels):
        for l, _ in levels:

            def grp_c(g, c2, l=l):
                for so in (0, 16):
                    o = g * 32 + so
                    pv = iota + o
                    pcol = pv * NCOL
                    fx = w_buf[l * 3, pl.ds(o, 16)]
                    fy = w_buf[l * 3 + 1, pl.ds(o, 16)]
                    fz = w_buf[l * 3 + 2, pl.ds(o, 16)]
                    gx, gy, gz = 1.0 - fx, 1.0 - fy, 1.0 - fz
                    wxy = (gx * gy, fx * gy, gx * fy, fx * fy)
                    acc0 = acc1 = None
                    for corner in range(8):
                        wc = wxy[corner & 3] * (fz if (corner & 4) else gz)
                        r4, oc = divmod(l * 8 + corner, 4)
                        if l < N_RES:
                            er = idx_buf[r4, pl.ds(oc * C + o, 16)]
                            ew = fv_buf[r4, pl.ds(oc * C + o, 16)]
                            f0 = plsc.load_gather(tloc, [er, ew])
                            f1 = plsc.load_gather(tloc, [er, ew + 1])
                        else:
                            rv = jnp.full((16,), r4, jnp.int32)
                            pv4 = pv + oc * C
                            fv = fv_buf[r4, pl.ds(oc * C + o, 16)]
                            f0 = plsc.load_gather(rows, [rv, pv4, fv])
                            f1 = plsc.load_gather(rows, [rv, pv4, fv + 1])
                        if corner == 0:
                            acc0, acc1 = f0 * wc, f1 * wc
                        else:
                            acc0, acc1 = acc0 + f0 * wc, acc1 + f1 * wc
                    plsc.store_scatter(obuf, [pcol + (3 + 2 * l)], acc0)
                    plsc.store_scatter(obuf, [pcol + (4 + 2 * l)], acc1)
                return c2

            lax.fori_loop(0, C // 32, grp_c, 0)

    def finish_xyz(pb):
        pbv = jnp.zeros((16,), jnp.int32) + pb

        def grp_x(g, c2):
            o = g * 16
            pv = iota + o
            p3 = pv * 3
            pcol = pv * NCOL
            x = plsc.load_gather(xyz_loc, [pbv, p3])
            y = plsc.load_gather(xyz_loc, [pbv, p3 + 1])
            z = plsc.load_gather(xyz_loc, [pbv, p3 + 2])
            plsc.store_scatter(obuf, [pcol], x * 2.0 - 1.0)
            plsc.store_scatter(obuf, [pcol + 1], y * 2.0 - 1.0)
            plsc.store_scatter(obuf, [pcol + 2], z * 2.0 - 1.0)
            return c2

        lax.fori_loop(0, C // 16, grp_x, 0)

    for l in range(N_RES):
        nrows = ((LEVELS[l][1] ** 3) * F + 7) // 8
        pltpu.sync_copy(tbl.at[pl.ds(l * T * F // 8, nrows)],
                        tloc.at[pl.ds(TLOC_BASE[l] // 8, nrows)])
    pltpu.sync_copy(xyzf.at[pl.ds(base0 * 3, C * 3)], xyz_loc.at[0])
    phase_a(H0, 0)

    def block(b, carry):
        pb = b & 1
        base = base0 + b * C
        bn = jnp.minimum(b + 1, nblk - 1)
        basen = base0 + bn * C

        pltpu.async_copy(
            xyzf.at[pl.ds(basen * 3, C * 3)], xyz_loc.at[1 - pb], sem_x)
        phase_a(H1, pb)
        finish_xyz(pb)
        phase_c(H0[:N_RES])
        drain_half(H0)
        phase_c(H0[N_RES:])
        pltpu.make_async_copy(
            xyzf.at[pl.ds(basen * 3, C * 3)], xyz_loc.at[1 - pb], sem_x).wait()
        phase_a(H0, 1 - pb)
        drain_half(H1)
        phase_c(H1)
        pltpu.sync_copy(obuf, out.at[pl.ds(base * NCOL, C * NCOL)])
        return carry

    lax.fori_loop(0, nblk, block, 0)

    drain_half(H0)


def kernel(xyz, table):
    n = xyz.shape[0]
    nw = N_LEVELS * T * F
    tnative = (table.reshape(N_LEVELS, T // 128, 128, F)
               .transpose(0, 1, 3, 2).reshape(nw))
    xyzf = xyz.reshape(n * 3)
    mesh = plsc.VectorSubcoreMesh(core_axis_name="c", subcore_axis_name="s")
    k1 = pl.kernel(
        _relayout_body,
        out_type=jax.ShapeDtypeStruct((nw,), jnp.float32),
        mesh=mesh,
        scratch_types=[
            pltpu.VMEM((RELAYOUT_CH,), jnp.float32),
            pltpu.VMEM((RELAYOUT_CH,), jnp.float32),
        ],
        compiler_params=pltpu.CompilerParams(
            needs_layout_passes=False, use_tc_tiling_on_sc=False),
    )
    tbl = k1(tnative).reshape(nw // 8, 8)
    k = pl.kernel(
        _body,
        out_type=jax.ShapeDtypeStruct((n * NCOL,), jnp.float32),
        mesh=mesh,
        scratch_types=[
            pltpu.VMEM((2, C * 3), jnp.float32),
            pltpu.VMEM((NR // 4, C * 4), jnp.int32),
            pltpu.VMEM((NR // 4, C * 4), jnp.int32),
            pltpu.VMEM((N_LEVELS * 3, C), jnp.float32),
            pltpu.VMEM((NR // 4, C * 4, 8), jnp.float32),
            pltpu.VMEM((C * NCOL,), jnp.float32),
            pltpu.VMEM((TLOC_SIZE // 8, 8), jnp.float32),
            pltpu.SemaphoreType.DMA((N_LEVELS,)),
            pltpu.SemaphoreType.DMA,
        ],
        compiler_params=pltpu.CompilerParams(
            needs_layout_passes=False, use_tc_tiling_on_sc=False),
    )
    return k(xyzf, tbl).reshape(n, NCOL)
